# baseline (speedup 1.0000x reference)
import functools
import math

import jax
import jax.numpy as jnp
from jax import lax
from jax.experimental import pallas as pl
from jax.experimental.pallas import tpu as pltpu
from jax.experimental.pallas import tpu_sc as plsc

N_WORKERS = 32
BLK = 3200
NBUF = 2
NEG_2PI2 = -2.0 * math.pi ** 2


@jax.jit
def _scaler_call(fcalc, s_t, bins, params):
    n = fcalc.shape[0]
    assert n % BLK == 0
    nblocks = n // BLK
    blocks_per_worker = -(-nblocks // N_WORKERS)
    assert blocks_per_worker % NBUF == 0

    mesh = plsc.VectorSubcoreMesh(core_axis_name="c", subcore_axis_name="s")

    scratch = []
    for _ in range(NBUF):
        scratch += [
            pltpu.VMEM((BLK,), jnp.float32),
            pltpu.VMEM((BLK,), jnp.int32),
            pltpu.VMEM((3, BLK), jnp.float32),
            pltpu.VMEM((BLK,), jnp.float32),
            pltpu.SemaphoreType.DMA,
            pltpu.SemaphoreType.DMA,
        ]
    scratch += [
        pltpu.VMEM((128,), jnp.float32),
    ]

    @functools.partial(
        pl.kernel,
        mesh=mesh,
        compiler_params=pltpu.CompilerParams(needs_layout_passes=False),
        out_type=jax.ShapeDtypeStruct((n,), jnp.float32),
        scratch_types=scratch,
    )
    def scaler_kernel(fcalc_hbm, st_hbm, bins_hbm, params_hbm, out_hbm, *bufs_flat):
        bufs = [bufs_flat[6 * i: 6 * i + 6] for i in range(NBUF)]
        prm_v = bufs_flat[6 * NBUF]
        wid = lax.axis_index("s") * 2 + lax.axis_index("c")

        pltpu.sync_copy(params_hbm, prm_v)
        prm_v[pl.ds(0, 16)] = jnp.exp(prm_v[pl.ds(0, 16)])
        prm_v[pl.ds(16, 16)] = jnp.exp(prm_v[pl.ds(16, 16)])

        a_xx = prm_v[pl.ds(32, 16)]
        a_yy = prm_v[pl.ds(48, 16)]
        a_zz = prm_v[pl.ds(64, 16)]
        a_xy = prm_v[pl.ds(80, 16)]
        a_xz = prm_v[pl.ds(96, 16)]
        a_yz = prm_v[pl.ds(112, 16)]

        def block_base(j):
            b = wid + N_WORKERS * j
            b = jnp.where(b < nblocks, b, b - N_WORKERS)
            return b * BLK

        def start_in(j, p):
            base = block_base(j)
            f_v, bi_v, s_v, _, isem, _ = bufs[p]
            pltpu.async_copy(fcalc_hbm.at[pl.ds(base, BLK)], f_v, isem)
            pltpu.async_copy(bins_hbm.at[pl.ds(base, BLK)], bi_v, isem)
            pltpu.async_copy(st_hbm.at[:, pl.ds(base, BLK)], s_v, isem)

        def wait_in(p):
            f_v, bi_v, s_v, _, isem, _ = bufs[p]
            pltpu.make_async_copy(fcalc_hbm.at[pl.ds(0, BLK)], f_v, isem).wait()
            pltpu.make_async_copy(bins_hbm.at[pl.ds(0, BLK)], bi_v, isem).wait()
            pltpu.make_async_copy(st_hbm.at[:, pl.ds(0, BLK)], s_v, isem).wait()

        def start_out(j, p):
            base = block_base(j)
            o_v, osem = bufs[p][3], bufs[p][5]
            pltpu.async_copy(o_v, out_hbm.at[pl.ds(base, BLK)], osem)

        def wait_out(p):
            o_v, osem = bufs[p][3], bufs[p][5]
            pltpu.make_async_copy(o_v, out_hbm.at[pl.ds(0, BLK)], osem).wait()

        def compute(p):
            f_v, bi_v, s_v, o_v = bufs[p][:4]

            @plsc.parallel_loop(0, BLK, 16, unroll=4)
            def body(off):
                x = s_v[0, pl.ds(off, 16)]
                y = s_v[1, pl.ds(off, 16)]
                z = s_v[2, pl.ds(off, 16)]
                scale = plsc.load_gather(prm_v, [bi_v[pl.ds(off, 16)]])
                f = f_v[pl.ds(off, 16)]
                t0 = a_xx * x + a_xy * y + a_xz * z
                t1 = a_yy * y + a_yz * z
                t2 = a_zz * z
                q = x * t0 + y * t1 + z * t2
                o_v[pl.ds(off, 16)] = f * scale * jnp.exp(q)

        for p in range(NBUF):
            start_in(p, p)

        @pl.loop(0, blocks_per_worker, step=NBUF)
        def outer(j0):
            for b in range(NBUF):
                jj = j0 + b
                wait_in(b)

                @pl.when(jj >= NBUF)
                def _():
                    wait_out(b)

                compute(b)
                start_out(jj, b)

                @pl.when(jj + NBUF < blocks_per_worker)
                def _():
                    start_in(jj + NBUF, b)

        for p in range(NBUF):
            wait_out(p)

    return scaler_kernel(fcalc, s_t, bins, params)


def kernel(fcalc, s, bins, log_scale, U):
    s_t = s.T
    scal = jnp.stack([U[0], U[1], U[2],
                      2.0 * U[3], 2.0 * U[4], 2.0 * U[5]]) * NEG_2PI2
    params = jnp.concatenate([
        log_scale,
        jnp.zeros((12,), jnp.float32),
        jnp.repeat(scal, 16),
    ])
    return _scaler_call(fcalc, s_t, bins.astype(jnp.int32), params)


# speedup vs baseline: 1.0768x; 1.0768x over previous
import functools
import math

import jax
import jax.numpy as jnp
from jax import lax
from jax.experimental import pallas as pl
from jax.experimental.pallas import tpu as pltpu
from jax.experimental.pallas import tpu_sc as plsc

N_WORKERS = 32
BLK = 3200
NBUF = 4
NEG_2PI2 = -2.0 * math.pi ** 2


@jax.jit
def _scaler_call(fcalc, s_t, bins, params):
    n = fcalc.shape[0]
    assert n % BLK == 0
    nblocks = n // BLK
    blocks_per_worker = -(-nblocks // N_WORKERS)
    assert blocks_per_worker % NBUF == 0

    mesh = plsc.VectorSubcoreMesh(core_axis_name="c", subcore_axis_name="s")

    scratch = []
    for _ in range(NBUF):
        scratch += [
            pltpu.VMEM((BLK,), jnp.float32),
            pltpu.VMEM((BLK,), jnp.int32),
            pltpu.VMEM((3, BLK), jnp.float32),
            pltpu.VMEM((BLK,), jnp.float32),
            pltpu.SemaphoreType.DMA,
            pltpu.SemaphoreType.DMA,
        ]
    scratch += [
        pltpu.VMEM((128,), jnp.float32),
    ]

    @functools.partial(
        pl.kernel,
        mesh=mesh,
        compiler_params=pltpu.CompilerParams(needs_layout_passes=False),
        out_type=jax.ShapeDtypeStruct((n,), jnp.float32),
        scratch_types=scratch,
    )
    def scaler_kernel(fcalc_hbm, st_hbm, bins_hbm, params_hbm, out_hbm, *bufs_flat):
        bufs = [bufs_flat[6 * i: 6 * i + 6] for i in range(NBUF)]
        prm_v = bufs_flat[6 * NBUF]
        wid = lax.axis_index("s") * 2 + lax.axis_index("c")

        pltpu.sync_copy(params_hbm, prm_v)
        prm_v[pl.ds(0, 16)] = jnp.exp(prm_v[pl.ds(0, 16)])
        prm_v[pl.ds(16, 16)] = jnp.exp(prm_v[pl.ds(16, 16)])

        a_xx = prm_v[pl.ds(32, 16)]
        a_yy = prm_v[pl.ds(48, 16)]
        a_zz = prm_v[pl.ds(64, 16)]
        a_xy = prm_v[pl.ds(80, 16)]
        a_xz = prm_v[pl.ds(96, 16)]
        a_yz = prm_v[pl.ds(112, 16)]

        def block_base(j):
            b = wid + N_WORKERS * j
            b = jnp.where(b < nblocks, b, b - N_WORKERS)
            return b * BLK

        def start_in(j, p):
            base = block_base(j)
            f_v, bi_v, s_v, _, isem, _ = bufs[p]
            pltpu.async_copy(fcalc_hbm.at[pl.ds(base, BLK)], f_v, isem)
            pltpu.async_copy(bins_hbm.at[pl.ds(base, BLK)], bi_v, isem)
            pltpu.async_copy(st_hbm.at[:, pl.ds(base, BLK)], s_v, isem)

        def wait_in(p):
            f_v, bi_v, s_v, _, isem, _ = bufs[p]
            pltpu.make_async_copy(fcalc_hbm.at[pl.ds(0, BLK)], f_v, isem).wait()
            pltpu.make_async_copy(bins_hbm.at[pl.ds(0, BLK)], bi_v, isem).wait()
            pltpu.make_async_copy(st_hbm.at[:, pl.ds(0, BLK)], s_v, isem).wait()

        def start_out(j, p):
            base = block_base(j)
            o_v, osem = bufs[p][3], bufs[p][5]
            pltpu.async_copy(o_v, out_hbm.at[pl.ds(base, BLK)], osem)

        def wait_out(p):
            o_v, osem = bufs[p][3], bufs[p][5]
            pltpu.make_async_copy(o_v, out_hbm.at[pl.ds(0, BLK)], osem).wait()

        def compute(p):
            f_v, bi_v, s_v, o_v = bufs[p][:4]

            @plsc.parallel_loop(0, BLK, 16, unroll=4)
            def body(off):
                x = s_v[0, pl.ds(off, 16)]
                y = s_v[1, pl.ds(off, 16)]
                z = s_v[2, pl.ds(off, 16)]
                scale = plsc.load_gather(prm_v, [bi_v[pl.ds(off, 16)]])
                f = f_v[pl.ds(off, 16)]
                t0 = a_xx * x + a_xy * y + a_xz * z
                t1 = a_yy * y + a_yz * z
                t2 = a_zz * z
                q = x * t0 + y * t1 + z * t2
                o_v[pl.ds(off, 16)] = f * scale * jnp.exp(q)

        for p in range(NBUF):
            start_in(p, p)

        @pl.loop(0, blocks_per_worker, step=NBUF)
        def outer(j0):
            for b in range(NBUF):
                jj = j0 + b
                wait_in(b)

                @pl.when(jj >= NBUF)
                def _():
                    wait_out(b)

                compute(b)
                start_out(jj, b)

                @pl.when(jj + NBUF < blocks_per_worker)
                def _():
                    start_in(jj + NBUF, b)

        for p in range(NBUF):
            wait_out(p)

    return scaler_kernel(fcalc, s_t, bins, params)


def kernel(fcalc, s, bins, log_scale, U):
    s_t = s.T
    scal = jnp.stack([U[0], U[1], U[2],
                      2.0 * U[3], 2.0 * U[4], 2.0 * U[5]]) * NEG_2PI2
    params = jnp.concatenate([
        log_scale,
        jnp.zeros((12,), jnp.float32),
        jnp.repeat(scal, 16),
    ])
    return _scaler_call(fcalc, s_t, bins.astype(jnp.int32), params)


# speedup vs baseline: 1.1585x; 1.0759x over previous
import functools
import math

import jax
import jax.numpy as jnp
from jax import lax
from jax.experimental import pallas as pl
from jax.experimental.pallas import tpu as pltpu
from jax.experimental.pallas import tpu_sc as plsc

N_WORKERS = 32
BLK = 3200
NBUF = 4
NEG_2PI2 = -2.0 * math.pi ** 2


@jax.jit
def _scaler_call(fcalc, s_t, bins, log_scale, U):
    n = fcalc.shape[0]
    assert n % BLK == 0
    nblocks = n // BLK
    blocks_per_worker = -(-nblocks // N_WORKERS)
    assert blocks_per_worker % NBUF == 0

    mesh = plsc.VectorSubcoreMesh(core_axis_name="c", subcore_axis_name="s")

    scratch = []
    for _ in range(NBUF):
        scratch += [
            pltpu.VMEM((BLK,), jnp.float32),
            pltpu.VMEM((BLK,), jnp.int32),
            pltpu.VMEM((3, BLK), jnp.float32),
            pltpu.VMEM((BLK,), jnp.float32),
            pltpu.SemaphoreType.DMA,
            pltpu.SemaphoreType.DMA,
        ]
    scratch += [
        pltpu.VMEM((32,), jnp.float32),
        pltpu.VMEM((32,), jnp.float32),
    ]

    @functools.partial(
        pl.kernel,
        mesh=mesh,
        compiler_params=pltpu.CompilerParams(needs_layout_passes=False),
        out_type=jax.ShapeDtypeStruct((n,), jnp.float32),
        scratch_types=scratch,
    )
    def scaler_kernel(fcalc_hbm, st_hbm, bins_hbm, ls_hbm, u_hbm, out_hbm,
                      *bufs_flat):
        bufs = [bufs_flat[6 * i: 6 * i + 6] for i in range(NBUF)]
        tab_v = bufs_flat[6 * NBUF]
        stg_v = bufs_flat[6 * NBUF + 1]
        wid = lax.axis_index("s") * 2 + lax.axis_index("c")

        stg_v[pl.ds(0, 16)] = jnp.zeros((16,), jnp.float32)
        stg_v[pl.ds(16, 16)] = jnp.zeros((16,), jnp.float32)
        pltpu.sync_copy(ls_hbm, stg_v.at[pl.ds(0, 20)])
        pltpu.sync_copy(u_hbm, stg_v.at[pl.ds(24, 6)])
        lo = stg_v[pl.ds(0, 16)]
        hi = stg_v[pl.ds(4, 16)]
        tab_v[pl.ds(0, 16)] = jnp.exp(lo)
        tab_v[pl.ds(4, 16)] = jnp.exp(hi)

        uvec = stg_v[pl.ds(16, 16)]
        lane = lax.iota(jnp.int32, 16)

        def ubcast(j, c):
            uj = jnp.sum(jnp.where(lane == j + 8, uvec, 0.0), axis=0)
            return lax.broadcast_in_dim(c * uj, (16,), ())

        a_xx = ubcast(0, NEG_2PI2)
        a_yy = ubcast(1, NEG_2PI2)
        a_zz = ubcast(2, NEG_2PI2)
        a_xy = ubcast(3, 2.0 * NEG_2PI2)
        a_xz = ubcast(4, 2.0 * NEG_2PI2)
        a_yz = ubcast(5, 2.0 * NEG_2PI2)

        def block_base(j):
            b = wid + N_WORKERS * j
            b = jnp.where(b < nblocks, b, b - N_WORKERS)
            return b * BLK

        def start_in(j, p):
            base = block_base(j)
            f_v, bi_v, s_v, _, isem, _ = bufs[p]
            pltpu.async_copy(fcalc_hbm.at[pl.ds(base, BLK)], f_v, isem)
            pltpu.async_copy(bins_hbm.at[pl.ds(base, BLK)], bi_v, isem)
            pltpu.async_copy(st_hbm.at[:, pl.ds(base, BLK)], s_v, isem)

        def wait_in(p):
            f_v, bi_v, s_v, _, isem, _ = bufs[p]
            pltpu.make_async_copy(fcalc_hbm.at[pl.ds(0, BLK)], f_v, isem).wait()
            pltpu.make_async_copy(bins_hbm.at[pl.ds(0, BLK)], bi_v, isem).wait()
            pltpu.make_async_copy(st_hbm.at[:, pl.ds(0, BLK)], s_v, isem).wait()

        def start_out(j, p):
            base = block_base(j)
            o_v, osem = bufs[p][3], bufs[p][5]
            pltpu.async_copy(o_v, out_hbm.at[pl.ds(base, BLK)], osem)

        def wait_out(p):
            o_v, osem = bufs[p][3], bufs[p][5]
            pltpu.make_async_copy(o_v, out_hbm.at[pl.ds(0, BLK)], osem).wait()

        def compute(p):
            f_v, bi_v, s_v, o_v = bufs[p][:4]

            @plsc.parallel_loop(0, BLK, 16, unroll=4)
            def body(off):
                x = s_v[0, pl.ds(off, 16)]
                y = s_v[1, pl.ds(off, 16)]
                z = s_v[2, pl.ds(off, 16)]
                scale = plsc.load_gather(tab_v, [bi_v[pl.ds(off, 16)]])
                f = f_v[pl.ds(off, 16)]
                t0 = a_xx * x + a_xy * y + a_xz * z
                t1 = a_yy * y + a_yz * z
                q = x * t0 + y * t1 + a_zz * (z * z)
                o_v[pl.ds(off, 16)] = f * scale * jnp.exp(q)

        for p in range(NBUF):
            start_in(p, p)

        @pl.loop(0, blocks_per_worker, step=NBUF)
        def outer(j0):
            for b in range(NBUF):
                jj = j0 + b
                wait_in(b)

                @pl.when(jj >= NBUF)
                def _():
                    wait_out(b)

                compute(b)
                start_out(jj, b)

                @pl.when(jj + NBUF < blocks_per_worker)
                def _():
                    start_in(jj + NBUF, b)

        for p in range(NBUF):
            wait_out(p)

    return scaler_kernel(fcalc, s_t, bins, log_scale, U)


def kernel(fcalc, s, bins, log_scale, U):
    return _scaler_call(fcalc, s.T, bins.astype(jnp.int32), log_scale, U)
